# Initial kernel scaffold; baseline (speedup 1.0000x reference)
import jax
import jax.numpy as jnp
from jax.experimental import pallas as pl


def kernel(hidden_states, W1, b1, W2, b2):
    raise NotImplementedError("write your pallas kernel here")


# speedup vs baseline: 4.2861x; 4.2861x over previous
import jax
import jax.numpy as jnp
from jax.experimental import pallas as pl
from jax.experimental.pallas import tpu as pltpu


_TOP_K = 2
_CAP_FACTOR = 1.5


def _router_block(x, w1, b1, w2, b2):
    E = w2.shape[1]
    h = jnp.maximum(jnp.dot(x, w1, preferred_element_type=jnp.float32) + b1, 0.0)
    logits = jnp.dot(h, w2, preferred_element_type=jnp.float32) + b2
    m = jnp.max(logits, axis=-1, keepdims=True)
    ex = jnp.exp(logits - m)
    probs = ex / jnp.sum(ex, axis=-1, keepdims=True)

    eidx = jax.lax.broadcasted_iota(jnp.int32, probs.shape, 1)
    m1 = jnp.max(probs, axis=-1, keepdims=True)
    i1 = jnp.min(jnp.where(probs == m1, eidx, E), axis=-1, keepdims=True)
    oh1 = eidx == i1
    pmasked = jnp.where(oh1, -1.0, probs)
    m2 = jnp.max(pmasked, axis=-1, keepdims=True)
    i2 = jnp.min(jnp.where(pmasked == m2, eidx, E), axis=-1, keepdims=True)
    oh2 = eidx == i2
    denom = m1 + m2
    comb_vals = jnp.where(oh1, m1 / denom, jnp.where(oh2, m2 / denom, 0.0))
    disp_vals = jnp.where(oh1 | oh2, 1.0, 0.0)
    return probs, disp_vals, comb_vals


def _tc_body(x_ref, w1_ref, b1_ref, w2_ref, b2_ref,
             disp_ref, comb_ref, probs_ref, aux_ref, acc_ref):
    i = pl.program_id(0)
    n = pl.num_programs(0)
    T = x_ref.shape[0]
    E = w2_ref.shape[1]
    CAP = disp_ref.shape[2]
    S_total = T * n

    probs, disp_vals, comb_vals = _router_block(
        x_ref[...], w1_ref[...], b1_ref[...], w2_ref[...], b2_ref[...])
    probs_ref[...] = probs

    cap0 = jax.lax.broadcasted_iota(jnp.int32, (T, E, 128), 2) == 0
    disp_ref[:, :, 0:128] = jnp.where(cap0, disp_vals[:, :, None], 0.0)
    comb_ref[:, :, 0:128] = jnp.where(cap0, comb_vals[:, :, None], 0.0)
    z = jnp.zeros((T, E, CAP - 128), dtype=jnp.float32)
    disp_ref[:, :, 128:CAP] = z
    comb_ref[:, :, 128:CAP] = z

    @pl.when(i == 0)
    def _():
        acc_ref[...] = jnp.zeros_like(acc_ref)

    acc_ref[...] += jnp.sum(probs, axis=0, keepdims=True)

    @pl.when(i == n - 1)
    def _():
        rppe = acc_ref[...] / S_total
        aux = jnp.sum(rppe * jnp.log(rppe * E + 1e-9))
        aux_ref[...] = jnp.full((1, 1), aux, dtype=jnp.float32)


def kernel(hidden_states, W1, b1, W2, b2):
    B, S, H = hidden_states.shape
    E = W2.shape[1]
    capacity = int(B * S * _CAP_FACTOR * _TOP_K / E)
    T = 256
    grid = (B * S) // T

    x = hidden_states.reshape(B * S, H)
    b1r = b1.reshape(1, H)
    b2r = b2.reshape(1, E)

    disp, comb, probs, aux = pl.pallas_call(
        _tc_body,
        grid=(grid,),
        in_specs=[
            pl.BlockSpec((T, H), lambda i: (i, 0)),
            pl.BlockSpec((H, H), lambda i: (0, 0)),
            pl.BlockSpec((1, H), lambda i: (0, 0)),
            pl.BlockSpec((H, E), lambda i: (0, 0)),
            pl.BlockSpec((1, E), lambda i: (0, 0)),
        ],
        out_specs=[
            pl.BlockSpec((T, E, capacity), lambda i: (i, 0, 0)),
            pl.BlockSpec((T, E, capacity), lambda i: (i, 0, 0)),
            pl.BlockSpec((T, E), lambda i: (i, 0)),
            pl.BlockSpec((1, 1), lambda i: (0, 0)),
        ],
        out_shape=[
            jax.ShapeDtypeStruct((B * S, E, capacity), jnp.float32),
            jax.ShapeDtypeStruct((B * S, E, capacity), jnp.float32),
            jax.ShapeDtypeStruct((B * S, E), jnp.float32),
            jax.ShapeDtypeStruct((1, 1), jnp.float32),
        ],
        scratch_shapes=[pltpu.VMEM((1, E), jnp.float32)],
        compiler_params=pltpu.CompilerParams(
            dimension_semantics=("arbitrary",),
        ),
    )(x, W1, b1r, W2, b2r)

    dispatch = disp.reshape(B, S, E, capacity)
    combine = comb.reshape(B, S, E, capacity)
    router_probs = probs.reshape(B, S, E)
    aux_loss = aux.reshape(())
    return (dispatch, combine, router_probs, aux_loss)
